# baseline (speedup 1.0000x reference)
import functools

import jax
import jax.numpy as jnp
import numpy as np
from jax import lax
from jax.experimental import pallas as pl
from jax.experimental.pallas import tpu as pltpu
from jax.experimental.pallas import tpu_sc as plsc

N = 10000
E = 320000
F_IN = 128
H = 8
HID = 16
NC = 64

_NUM_SC = 2
_NUM_SUB = 16
_TILES = _NUM_SC * _NUM_SUB
_EPT = E // _TILES
_CHUNK = 80
_NCH = _EPT // _CHUNK
_NPAD = 10240
_RPS = _NPAD // _NUM_SUB
_CB = 80
_BN_SCALE = float(1.0 / np.sqrt(1.0 + 1e-5))


def _bcast_lane(vec, j):
  return jnp.broadcast_to(vec[j], (16,))


def _make_edge_kernel(heads):
  hw = heads * HID if heads > 1 else NC
  rw = hw + 16
  nh = hw // 16
  mesh = plsc.VectorSubcoreMesh(core_axis_name="c", subcore_axis_name="s")

  @functools.partial(
      pl.kernel,
      out_type=jax.ShapeDtypeStruct((2 * _NPAD, rw), jnp.float32),
      mesh=mesh,
      compiler_params=pltpu.CompilerParams(use_tc_tiling_on_sc=False),
      scratch_types=[
          pltpu.VMEM((_CHUNK,), jnp.int32),
          pltpu.VMEM((_CHUNK,), jnp.int32),
          pltpu.VMEM((_CHUNK,), jnp.int32),
          pltpu.VMEM((_CHUNK,), jnp.int32),
          pltpu.VMEM((_CHUNK, rw), jnp.float32),
          pltpu.VMEM((_CHUNK, rw), jnp.float32),
          pltpu.VMEM((_CHUNK, 16), jnp.float32),
          pltpu.VMEM((_CHUNK, 16), jnp.float32),
          pltpu.VMEM((_CHUNK, rw), jnp.float32),
          pltpu.VMEM((48,), jnp.int32),
          pltpu.VMEM((32,), jnp.int32),
          pltpu.VMEM_SHARED((_NPAD, rw), jnp.float32),
          pltpu.SemaphoreType.DMA,
          pltpu.SemaphoreType.DMA,
          pltpu.SemaphoreType.DMA,
          pltpu.SemaphoreType.DMA,
          pltpu.SemaphoreType.DMA,
          pltpu.SemaphoreType.DMA,
          pltpu.SemaphoreType.DMA,
          pltpu.SemaphoreType.DMA,
          pltpu.SemaphoreType.DMA,
      ],
  )
  def edge_kernel(hext, adstp, sidx, didx, out, sb0, sb1, db0, db1,
                  r1a, r1b, r2a, r2b, contrib, dlo, dhi, acc,
                  x1a, x1b, x2a, x2b, i1a, i1b, i2a, i2b, xsc):
    cid = lax.axis_index("c")
    sid = lax.axis_index("s")
    wid = cid * _NUM_SUB + sid
    e0 = wid * _EPT
    base = sid * _RPS
    bufs = ((sb0, db0, r1a, r2a, x1a, x2a, i1a, i2a),
            (sb1, db1, r1b, r2b, x1b, x2b, i1b, i2b))

    def launch_idx(g, b):
      sb, db, r1, r2, x1, x2, i1, i2 = bufs[b]
      off = e0 + g * _CHUNK
      pltpu.async_copy(sidx.at[pl.ds(off, _CHUNK)], sb, i1)
      pltpu.async_copy(didx.at[pl.ds(off, _CHUNK)], db, i2)

    def issue(g, b):
      sb, db, r1, r2, x1, x2, i1, i2 = bufs[b]
      off = e0 + g * _CHUNK
      pltpu.make_async_copy(sidx.at[pl.ds(off, _CHUNK)], sb, i1).wait()
      pltpu.make_async_copy(didx.at[pl.ds(off, _CHUNK)], db, i2).wait()
      pltpu.async_copy(hext.at[sb], r1, x1)
      pltpu.async_copy(adstp.at[db], r2, x2)

    def consume(g, b):
      sb, db, r1, r2, x1, x2, i1, i2 = bufs[b]
      pltpu.make_async_copy(hext.at[sb], r1, x1).wait()
      pltpu.make_async_copy(adstp.at[db], r2, x2).wait()

      for i in range(3):
        dlo[pl.ds(i * 16, 16)] = db[pl.ds(i * 16, 16)]
      for i in range(2):
        dhi[pl.ds(i * 16, 16)] = db[pl.ds(48 + i * 16, 16)]

      @pl.when(g + 2 < _NCH)
      def _():
        launch_idx(g + 2, b)

      def edge_body(k):
        av = r1[k, pl.ds(hw, 16)]
        bv = r2[k, pl.ds(0, 16)]
        xv = av + bv
        xv = jnp.where(xv >= 0.0, xv, 0.2 * xv)
        exv = jnp.exp(xv)
        contrib[k, pl.ds(hw, 16)] = exv
        for j in range(nh):
          eb = _bcast_lane(exv, j if heads > 1 else 0)
          contrib[k, pl.ds(j * 16, 16)] = r1[k, pl.ds(j * 16, 16)] * eb

      plsc.parallel_loop(0, 48, step=1, unroll=4)(edge_body)
      pltpu.async_copy(contrib.at[pl.ds(0, 48)], acc.at[dlo], xsc, add=True)
      plsc.parallel_loop(48, _CHUNK, step=1, unroll=4)(edge_body)
      pltpu.sync_copy(contrib.at[pl.ds(48, _CHUNK - 48)], acc.at[dhi], add=True)
      pltpu.make_async_copy(contrib.at[pl.ds(0, 48)], acc.at[dlo], xsc).wait()

    launch_idx(0, 0)
    launch_idx(1, 1)
    issue(0, 0)

    zv = jnp.zeros((16,), jnp.float32)

    def zrow(i, _):
      def zcol(j, _):
        contrib[i, pl.ds(j * 16, 16)] = zv
        return 0
      return lax.fori_loop(0, rw // 16, zcol, 0)

    lax.fori_loop(0, _CB, zrow, 0)
    for i in range(_RPS // _CB):
      pltpu.sync_copy(contrib, acc.at[pl.ds(base + i * _CB, _CB)])
    plsc.subcore_barrier()

    def pair(p, _):
      issue(2 * p + 1, 1)
      consume(2 * p, 0)
      issue(2 * p + 2, 0)
      consume(2 * p + 1, 1)
      return 0

    lax.fori_loop(0, (_NCH - 1) // 2, pair, 0)
    consume(_NCH - 1, 0)

    plsc.subcore_barrier()
    for i in range(_RPS // _CB):
      r0 = base + i * _CB
      pltpu.async_copy(acc.at[pl.ds(r0, _CB)],
                       out.at[pl.ds(cid * _NPAD + r0, _CB)], xsc)
    for i in range(_RPS // _CB):
      r0 = base + i * _CB
      pltpu.make_async_copy(acc.at[pl.ds(r0, _CB)],
                            out.at[pl.ds(cid * _NPAD + r0, _CB)], xsc).wait()

  return edge_kernel


_edge_kernel_8 = _make_edge_kernel(H)
_edge_kernel_1 = _make_edge_kernel(1)


_BLK = 2000


def _mm_body(x_ref, w_ref, hext_ref, adst_ref, *, rw):
  y = jnp.dot(x_ref[...], w_ref[...], preferred_element_type=jnp.float32)
  hext_ref[...] = y[:, :rw]
  adst_ref[...] = y[:, rw:rw + 16]


def _mm_call(x, wcat, rw):
  wtot = wcat.shape[1]
  return pl.pallas_call(
      functools.partial(_mm_body, rw=rw),
      grid=(N // _BLK,),
      in_specs=[
          pl.BlockSpec((_BLK, x.shape[1]), lambda i: (i, 0)),
          pl.BlockSpec((x.shape[1], wtot), lambda i: (0, 0)),
      ],
      out_specs=[
          pl.BlockSpec((_BLK, rw), lambda i: (i, 0)),
          pl.BlockSpec((_BLK, 16), lambda i: (i, 0)),
      ],
      out_shape=[
          jax.ShapeDtypeStruct((N, rw), jnp.float32),
          jax.ShapeDtypeStruct((N, 16), jnp.float32),
      ],
  )(x, wcat)


def _fin_body(acc_ref, emat_ref, w_ref, gbb_ref, hext_ref, adst_ref, *, rw):
  s = acc_ref[0] + acc_ref[1]
  den = jnp.dot(s[:, 128:136], emat_ref[...], preferred_element_type=jnp.float32)
  h = s[:, :128] / (den + 1e-16) + gbb_ref[2:3, :]
  h = jnp.where(h > 0.0, h, jnp.exp(h) - 1.0)
  h = h * gbb_ref[0:1, :] + gbb_ref[1:2, :]
  y = jnp.dot(h, w_ref[...], preferred_element_type=jnp.float32)
  hext_ref[...] = y[:, :rw]
  adst_ref[...] = y[:, rw:rw + 16]


_FBLK = 2048


def _fin_call(acc, emat, wcat, gbb, rw):
  wtot = wcat.shape[1]
  return pl.pallas_call(
      functools.partial(_fin_body, rw=rw),
      grid=(_NPAD // _FBLK,),
      in_specs=[
          pl.BlockSpec((2, _FBLK, 144), lambda i: (0, i, 0)),
          pl.BlockSpec((8, 128), lambda i: (0, 0)),
          pl.BlockSpec((128, wtot), lambda i: (0, 0)),
          pl.BlockSpec((3, 128), lambda i: (0, 0)),
      ],
      out_specs=[
          pl.BlockSpec((_FBLK, rw), lambda i: (i, 0)),
          pl.BlockSpec((_FBLK, 16), lambda i: (i, 0)),
      ],
      out_shape=[
          jax.ShapeDtypeStruct((_NPAD, rw), jnp.float32),
          jax.ShapeDtypeStruct((_NPAD, 16), jnp.float32),
      ],
  )(acc, emat, wcat, gbb)


def _fin3_body(acc_ref, emat_ref, b_ref, out_ref):
  s = acc_ref[0] + acc_ref[1]
  den = jnp.dot(s[:, 64:80], emat_ref[...], preferred_element_type=jnp.float32)
  h = s[:, :64] / (den + 1e-16) + b_ref[0:1, :]
  m = jnp.max(h, axis=1, keepdims=True)
  ex = jnp.exp(h - m)
  lse = jnp.log(jnp.sum(ex, axis=1, keepdims=True))
  out_ref[...] = h - m - lse


def _fin3_call(acc, emat3, b3):
  return pl.pallas_call(
      _fin3_body,
      grid=(_NPAD // _FBLK,),
      in_specs=[
          pl.BlockSpec((2, _FBLK, 80), lambda i: (0, i, 0)),
          pl.BlockSpec((16, 64), lambda i: (0, 0)),
          pl.BlockSpec((1, 64), lambda i: (0, 0)),
      ],
      out_specs=pl.BlockSpec((_FBLK, 64), lambda i: (i, 0)),
      out_shape=jax.ShapeDtypeStruct((_NPAD, 64), jnp.float32),
  )(acc, emat3, b3)


def _wcat(W, a_s, a_d, heads, out_ch):
  f = W.shape[0]
  hw = heads * out_ch
  wr = W.reshape(f, heads, out_ch)
  ws = jnp.einsum("fhc,hc->fh", wr, a_s)
  wd = jnp.einsum("fhc,hc->fh", wr, a_d)
  pad = jnp.zeros((f, 16 - heads), jnp.float32)
  return jnp.concatenate([W, ws, pad, wd, pad], axis=1)


def kernel(x, edge_index, W1, a1s, a1d, b1, g1, be1,
           W2, a2s, a2d, b2, g2, be2, W3, a3s, a3d, b3):
  sidx = edge_index[0]
  didx = edge_index[1]

  wcat1 = _wcat(W1, a1s, a1d, H, HID)
  wcat2 = _wcat(W2, a2s, a2d, H, HID)
  wcat3 = _wcat(W3, a3s, a3d, 1, NC)

  emat = jnp.asarray(np.kron(np.eye(8, dtype=np.float32),
                             np.ones((1, 16), np.float32)))
  e3 = np.zeros((16, 64), np.float32)
  e3[0, :] = 1.0
  emat3 = jnp.asarray(e3)

  def gbb_rows(g, be, b):
    return jnp.stack([g * _BN_SCALE, be, b])

  hext1, adst1 = _mm_call(x, wcat1, 144)
  acc1 = _edge_kernel_8(hext1, adst1, sidx, didx).reshape(2, _NPAD, 144)

  hext2, adst2 = _fin_call(acc1, emat, wcat2, gbb_rows(g1, be1, b1), 144)
  acc2 = _edge_kernel_8(hext2, adst2, sidx, didx).reshape(2, _NPAD, 144)

  hext3, adst3 = _fin_call(acc2, emat, wcat3, gbb_rows(g2, be2, b2), 80)
  acc3 = _edge_kernel_1(hext3, adst3, sidx, didx).reshape(2, _NPAD, 80)

  return _fin3_call(acc3, emat3, b3.reshape(1, 64))[:N]


# speedup vs baseline: 1.0861x; 1.0861x over previous
import functools

import jax
import jax.numpy as jnp
import numpy as np
from jax import lax
from jax.experimental import pallas as pl
from jax.experimental.pallas import tpu as pltpu
from jax.experimental.pallas import tpu_sc as plsc

N = 10000
E = 320000
F_IN = 128
H = 8
HID = 16
NC = 64

_NUM_SC = 2
_NUM_SUB = 16
_TILES = _NUM_SC * _NUM_SUB
_EPT = E // _TILES
_CHUNK = 80
_NCH = _EPT // _CHUNK
_NPAD = 10240
_RPS = _NPAD // _NUM_SUB
_CB = 80
_BN_SCALE = float(1.0 / np.sqrt(1.0 + 1e-5))


def _bcast_lane(vec, j):
  return jnp.broadcast_to(vec[j], (16,))


def _make_edge_kernel(heads):
  hw = heads * HID if heads > 1 else NC
  rw = hw + 16
  nh = hw // 16
  mesh = plsc.VectorSubcoreMesh(core_axis_name="c", subcore_axis_name="s")

  @functools.partial(
      pl.kernel,
      out_type=jax.ShapeDtypeStruct((2 * _NPAD, rw), jnp.float32),
      mesh=mesh,
      compiler_params=pltpu.CompilerParams(use_tc_tiling_on_sc=False,
                                           needs_layout_passes=False),
      scratch_types=[
          pltpu.VMEM((_CHUNK,), jnp.int32),
          pltpu.VMEM((_CHUNK,), jnp.int32),
          pltpu.VMEM((_CHUNK,), jnp.int32),
          pltpu.VMEM((_CHUNK,), jnp.int32),
          pltpu.VMEM((_CHUNK, hw), jnp.bfloat16),
          pltpu.VMEM((_CHUNK, hw), jnp.bfloat16),
          pltpu.VMEM((_CHUNK, 16), jnp.float32),
          pltpu.VMEM((_CHUNK, 16), jnp.float32),
          pltpu.VMEM((_CHUNK, 16), jnp.float32),
          pltpu.VMEM((_CHUNK, 16), jnp.float32),
          pltpu.VMEM((_CHUNK, rw), jnp.float32),
          pltpu.VMEM((48,), jnp.int32),
          pltpu.VMEM((32,), jnp.int32),
          pltpu.VMEM_SHARED((_NPAD, rw), jnp.float32),
          pltpu.SemaphoreType.DMA,
          pltpu.SemaphoreType.DMA,
          pltpu.SemaphoreType.DMA,
          pltpu.SemaphoreType.DMA,
          pltpu.SemaphoreType.DMA,
          pltpu.SemaphoreType.DMA,
          pltpu.SemaphoreType.DMA,
          pltpu.SemaphoreType.DMA,
          pltpu.SemaphoreType.DMA,
          pltpu.SemaphoreType.DMA,
          pltpu.SemaphoreType.DMA,
      ],
  )
  def edge_kernel(hbf, asrcp, adstp, sidx, didx, out, sb0, sb1, db0, db1,
                  r1a, r1b, r2a, r2b, r3a, r3b, contrib, dlo, dhi, acc,
                  x1a, x1b, x2a, x2b, x3a, x3b, i1a, i1b, i2a, i2b, xsc):
    cid = lax.axis_index("c")
    sid = lax.axis_index("s")
    wid = cid * _NUM_SUB + sid
    e0 = wid * _EPT
    base = sid * _RPS
    bufs = ((sb0, db0, r1a, r2a, r3a, x1a, x2a, x3a, i1a, i2a),
            (sb1, db1, r1b, r2b, r3b, x1b, x2b, x3b, i1b, i2b))

    def launch_idx(g, b):
      sb, db, r1, r2, r3, x1, x2, x3, i1, i2 = bufs[b]
      off = e0 + g * _CHUNK
      pltpu.async_copy(sidx.at[pl.ds(off, _CHUNK)], sb, i1)
      pltpu.async_copy(didx.at[pl.ds(off, _CHUNK)], db, i2)

    def issue(g, b):
      sb, db, r1, r2, r3, x1, x2, x3, i1, i2 = bufs[b]
      off = e0 + g * _CHUNK
      pltpu.make_async_copy(sidx.at[pl.ds(off, _CHUNK)], sb, i1).wait()
      pltpu.make_async_copy(didx.at[pl.ds(off, _CHUNK)], db, i2).wait()
      pltpu.async_copy(hbf.at[sb], r1, x1)
      pltpu.async_copy(asrcp.at[sb], r3, x3)
      pltpu.async_copy(adstp.at[db], r2, x2)

    def consume(g, b):
      sb, db, r1, r2, r3, x1, x2, x3, i1, i2 = bufs[b]
      pltpu.make_async_copy(hbf.at[sb], r1, x1).wait()
      pltpu.make_async_copy(asrcp.at[sb], r3, x3).wait()
      pltpu.make_async_copy(adstp.at[db], r2, x2).wait()

      for i in range(3):
        dlo[pl.ds(i * 16, 16)] = db[pl.ds(i * 16, 16)]
      for i in range(2):
        dhi[pl.ds(i * 16, 16)] = db[pl.ds(48 + i * 16, 16)]

      @pl.when(g + 2 < _NCH)
      def _():
        launch_idx(g + 2, b)

      def edge_body(k):
        av = r3[k, pl.ds(0, 16)]
        bv = r2[k, pl.ds(0, 16)]
        xv = av + bv
        xv = jnp.where(xv >= 0.0, xv, 0.2 * xv)
        exv = jnp.exp(xv)
        contrib[k, pl.ds(hw, 16)] = exv
        for m in range(nh // 2):
          ab = r1[k, pl.ds(m * 32, 32)]
          a_, b_ = plsc.unpack(ab, format=plsc.PackFormat.INTERLEAVED,
                               preferred_element_type=jnp.float32)
          ea = _bcast_lane(exv, 2 * m if heads > 1 else 0)
          eb = _bcast_lane(exv, 2 * m + 1 if heads > 1 else 0)
          contrib[k, pl.ds(m * 32, 16)] = a_ * ea
          contrib[k, pl.ds(m * 32 + 16, 16)] = b_ * eb

      plsc.parallel_loop(0, 48, step=1, unroll=4)(edge_body)
      pltpu.async_copy(contrib.at[pl.ds(0, 48)], acc.at[dlo], xsc, add=True)
      plsc.parallel_loop(48, _CHUNK, step=1, unroll=4)(edge_body)
      pltpu.sync_copy(contrib.at[pl.ds(48, _CHUNK - 48)], acc.at[dhi], add=True)
      pltpu.make_async_copy(contrib.at[pl.ds(0, 48)], acc.at[dlo], xsc).wait()

    launch_idx(0, 0)
    launch_idx(1, 1)
    issue(0, 0)

    zv = jnp.zeros((16,), jnp.float32)

    def zrow(i, _):
      def zcol(j, _):
        contrib[i, pl.ds(j * 16, 16)] = zv
        return 0
      return lax.fori_loop(0, rw // 16, zcol, 0)

    lax.fori_loop(0, _CB, zrow, 0)
    for i in range(_RPS // _CB):
      pltpu.sync_copy(contrib, acc.at[pl.ds(base + i * _CB, _CB)])
    plsc.subcore_barrier()

    def pair(p, _):
      issue(2 * p + 1, 1)
      consume(2 * p, 0)
      issue(2 * p + 2, 0)
      consume(2 * p + 1, 1)
      return 0

    lax.fori_loop(0, (_NCH - 1) // 2, pair, 0)
    consume(_NCH - 1, 0)

    plsc.subcore_barrier()
    for i in range(_RPS // _CB):
      r0 = base + i * _CB
      pltpu.async_copy(acc.at[pl.ds(r0, _CB)],
                       out.at[pl.ds(cid * _NPAD + r0, _CB)], xsc)
    for i in range(_RPS // _CB):
      r0 = base + i * _CB
      pltpu.make_async_copy(acc.at[pl.ds(r0, _CB)],
                            out.at[pl.ds(cid * _NPAD + r0, _CB)], xsc).wait()

  return edge_kernel


_edge_kernel_8 = _make_edge_kernel(H)
_edge_kernel_1 = _make_edge_kernel(1)


_BLK = 2000


def _mm_body(x_ref, w_ref, hbf_ref, asrc_ref, adst_ref, *, hw):
  y = jnp.dot(x_ref[...], w_ref[...], preferred_element_type=jnp.float32)
  hbf_ref[...] = y[:, :hw].astype(jnp.bfloat16)
  asrc_ref[...] = y[:, hw:hw + 16]
  adst_ref[...] = y[:, hw + 16:hw + 32]


def _mm_call(x, wcat, hw):
  wtot = wcat.shape[1]
  return pl.pallas_call(
      functools.partial(_mm_body, hw=hw),
      grid=(N // _BLK,),
      in_specs=[
          pl.BlockSpec((_BLK, x.shape[1]), lambda i: (i, 0)),
          pl.BlockSpec((x.shape[1], wtot), lambda i: (0, 0)),
      ],
      out_specs=[
          pl.BlockSpec((_BLK, hw), lambda i: (i, 0)),
          pl.BlockSpec((_BLK, 16), lambda i: (i, 0)),
          pl.BlockSpec((_BLK, 16), lambda i: (i, 0)),
      ],
      out_shape=[
          jax.ShapeDtypeStruct((N, hw), jnp.bfloat16),
          jax.ShapeDtypeStruct((N, 16), jnp.float32),
          jax.ShapeDtypeStruct((N, 16), jnp.float32),
      ],
  )(x, wcat)


def _fin_body(acc_ref, emat_ref, w_ref, gbb_ref, hbf_ref, asrc_ref, adst_ref,
              *, hw):
  s = acc_ref[0] + acc_ref[1]
  den = jnp.dot(s[:, 128:136], emat_ref[...], preferred_element_type=jnp.float32)
  h = s[:, :128] / (den + 1e-16) + gbb_ref[2:3, :]
  h = jnp.where(h > 0.0, h, jnp.exp(h) - 1.0)
  h = h * gbb_ref[0:1, :] + gbb_ref[1:2, :]
  y = jnp.dot(h, w_ref[...], preferred_element_type=jnp.float32)
  hbf_ref[...] = y[:, :hw].astype(jnp.bfloat16)
  asrc_ref[...] = y[:, hw:hw + 16]
  adst_ref[...] = y[:, hw + 16:hw + 32]


_FBLK = 2048


def _fin_call(acc, emat, wcat, gbb, hw):
  wtot = wcat.shape[1]
  return pl.pallas_call(
      functools.partial(_fin_body, hw=hw),
      grid=(_NPAD // _FBLK,),
      in_specs=[
          pl.BlockSpec((2, _FBLK, 144), lambda i: (0, i, 0)),
          pl.BlockSpec((8, 128), lambda i: (0, 0)),
          pl.BlockSpec((128, wtot), lambda i: (0, 0)),
          pl.BlockSpec((3, 128), lambda i: (0, 0)),
      ],
      out_specs=[
          pl.BlockSpec((_FBLK, hw), lambda i: (i, 0)),
          pl.BlockSpec((_FBLK, 16), lambda i: (i, 0)),
          pl.BlockSpec((_FBLK, 16), lambda i: (i, 0)),
      ],
      out_shape=[
          jax.ShapeDtypeStruct((_NPAD, hw), jnp.bfloat16),
          jax.ShapeDtypeStruct((_NPAD, 16), jnp.float32),
          jax.ShapeDtypeStruct((_NPAD, 16), jnp.float32),
      ],
  )(acc, emat, wcat, gbb)


def _fin3_body(acc_ref, emat_ref, b_ref, out_ref):
  s = acc_ref[0] + acc_ref[1]
  den = jnp.dot(s[:, 64:80], emat_ref[...], preferred_element_type=jnp.float32)
  h = s[:, :64] / (den + 1e-16) + b_ref[0:1, :]
  m = jnp.max(h, axis=1, keepdims=True)
  ex = jnp.exp(h - m)
  lse = jnp.log(jnp.sum(ex, axis=1, keepdims=True))
  out_ref[...] = h - m - lse


def _fin3_call(acc, emat3, b3):
  return pl.pallas_call(
      _fin3_body,
      grid=(_NPAD // _FBLK,),
      in_specs=[
          pl.BlockSpec((2, _FBLK, 80), lambda i: (0, i, 0)),
          pl.BlockSpec((16, 64), lambda i: (0, 0)),
          pl.BlockSpec((1, 64), lambda i: (0, 0)),
      ],
      out_specs=pl.BlockSpec((_FBLK, 64), lambda i: (i, 0)),
      out_shape=jax.ShapeDtypeStruct((_NPAD, 64), jnp.float32),
  )(acc, emat3, b3)


def _pair_perm(hw):
  perm = np.empty(hw, np.int64)
  for m in range(hw // 32):
    for t in range(16):
      perm[32 * m + 2 * t] = 32 * m + t
      perm[32 * m + 2 * t + 1] = 32 * m + 16 + t
  return perm


def _wcat(W, a_s, a_d, heads, out_ch):
  f = W.shape[0]
  hw = heads * out_ch
  wr = W.reshape(f, heads, out_ch)
  ws = jnp.einsum("fhc,hc->fh", wr, a_s)
  wd = jnp.einsum("fhc,hc->fh", wr, a_d)
  pad = jnp.zeros((f, 16 - heads), jnp.float32)
  return jnp.concatenate([W[:, _pair_perm(hw)], ws, pad, wd, pad], axis=1)


def kernel(x, edge_index, W1, a1s, a1d, b1, g1, be1,
           W2, a2s, a2d, b2, g2, be2, W3, a3s, a3d, b3):
  sidx = edge_index[0]
  didx = edge_index[1]

  wcat1 = _wcat(W1, a1s, a1d, H, HID)
  wcat2 = _wcat(W2, a2s, a2d, H, HID)
  wcat3 = _wcat(W3, a3s, a3d, 1, NC)

  emat = jnp.asarray(np.kron(np.eye(8, dtype=np.float32),
                             np.ones((1, 16), np.float32)))
  e3 = np.zeros((16, 64), np.float32)
  e3[0, :] = 1.0
  emat3 = jnp.asarray(e3)

  def gbb_rows(g, be, b):
    return jnp.stack([g * _BN_SCALE, be, b])

  hbf1, asrc1, adst1 = _mm_call(x, wcat1, 128)
  acc1 = _edge_kernel_8(hbf1, asrc1, adst1, sidx, didx).reshape(2, _NPAD, 144)

  hbf2, asrc2, adst2 = _fin_call(acc1, emat, wcat2, gbb_rows(g1, be1, b1), 128)
  acc2 = _edge_kernel_8(hbf2, asrc2, adst2, sidx, didx).reshape(2, _NPAD, 144)

  hbf3, asrc3, adst3 = _fin_call(acc2, emat, wcat3, gbb_rows(g2, be2, b2), 64)
  acc3 = _edge_kernel_1(hbf3, asrc3, adst3, sidx, didx).reshape(2, _NPAD, 80)

  return _fin3_call(acc3, emat3, b3.reshape(1, 64))[:N]
